# Initial kernel scaffold; baseline (speedup 1.0000x reference)
import jax
import jax.numpy as jnp
from jax.experimental import pallas as pl


def kernel(tokens, vocab):
    raise NotImplementedError("write your pallas kernel here")


# speedup vs baseline: 1650.8144x; 1650.8144x over previous
import functools

import jax
import jax.numpy as jnp
from jax import lax
from jax.experimental import pallas as pl
from jax.experimental.pallas import tpu as pltpu
from jax.experimental.pallas import tpu_sc as plsc

TOKEN_UNIVERSE = 120000
NUM_OOV = 1
NUM_WORKERS = 32
LANES = 16
CHUNK = 2048
VCHUNK = 2000


def _sc_lookup(tok32, voc32):
    n = tok32.shape[0]
    v = voc32.shape[0]
    per_w = n // NUM_WORKERS
    n_chunks = per_w // CHUNK
    v_chunks = v // VCHUNK
    assert per_w * NUM_WORKERS == n and n_chunks * CHUNK == per_w
    assert v_chunks * VCHUNK == v

    mesh = plsc.VectorSubcoreMesh(
        core_axis_name="c", subcore_axis_name="s", num_cores=2, num_subcores=16
    )

    @functools.partial(
        pl.kernel,
        out_type=jax.ShapeDtypeStruct((n,), jnp.int32),
        mesh=mesh,
        compiler_params=pltpu.CompilerParams(needs_layout_passes=False),
        scratch_types=[
            pltpu.VMEM((TOKEN_UNIVERSE,), jnp.int32),
            pltpu.VMEM((VCHUNK,), jnp.int32),
            pltpu.VMEM((CHUNK,), jnp.int32),
            pltpu.VMEM((CHUNK,), jnp.int32),
        ],
    )
    def k(tok_hbm, voc_hbm, out_hbm, table_v, vbuf_v, inb_v, outb_v):
        lane = lax.iota(jnp.int32, LANES)

        def zero_body(i, _):
            table_v[pl.ds(i * LANES, LANES)] = jnp.zeros((LANES,), jnp.int32)
            return _

        lax.fori_loop(jnp.int32(0), jnp.int32(TOKEN_UNIVERSE // LANES),
                      zero_body, None)

        def build_chunk(c, _):
            pltpu.sync_copy(voc_hbm.at[pl.ds(c * VCHUNK, VCHUNK)], vbuf_v)

            def scatter_body(j, _):
                ids = vbuf_v[pl.ds(j * LANES, LANES)]
                vals = c * VCHUNK + j * LANES + NUM_OOV + lane
                plsc.store_scatter(table_v, [ids], vals)
                return _

            lax.fori_loop(jnp.int32(0), jnp.int32(VCHUNK // LANES),
                          scatter_body, None)
            return _

        lax.fori_loop(jnp.int32(0), jnp.int32(v_chunks), build_chunk, None)

        wid = lax.axis_index("s") * 2 + lax.axis_index("c")
        base = wid * jnp.int32(per_w)

        def lookup_chunk(c, _):
            off = base + c * CHUNK
            pltpu.sync_copy(tok_hbm.at[pl.ds(off, CHUNK)], inb_v)

            def gather_body(j, _):
                t = inb_v[pl.ds(j * LANES, LANES)]
                outb_v[pl.ds(j * LANES, LANES)] = plsc.load_gather(table_v, [t])
                return _

            lax.fori_loop(jnp.int32(0), jnp.int32(CHUNK // LANES),
                          gather_body, None)
            pltpu.sync_copy(outb_v, out_hbm.at[pl.ds(off, CHUNK)])
            return _

        lax.fori_loop(jnp.int32(0), jnp.int32(n_chunks), lookup_chunk, None)

    return k(tok32, voc32)


def kernel(tokens, vocab):
    tok32 = tokens.astype(jnp.int32).reshape(-1)
    voc32 = vocab.astype(jnp.int32)
    out32 = _sc_lookup(tok32, voc32)
    return out32.reshape(tokens.shape).astype(tokens.dtype)
